# baseline (speedup 1.0000x reference)
import functools

import jax
import jax.numpy as jnp
from jax.experimental import pallas as pl

B, N, M = 4, 16384, 1024
C_FEAT = 64
C_PREV = 64
C_OUT = 64

TILE_N = 512
INF = 3.0e38


def _fused_body(xyzt_ref, xyzp_ref, feat_ref, fp_ref, w1a_ref, w1b_ref,
                b1_ref, out_ref):
    q = xyzt_ref[0]
    k = xyzp_ref[0]

    d2 = jnp.zeros((M, TILE_N), jnp.float32)
    for c in range(3):
        diff = k[:, c:c + 1] - q[c:c + 1, :]
        d2 = d2 + diff * diff

    NP = M // 2
    iota2 = jax.lax.broadcasted_iota(jnp.int32, (M, TILE_N), 0) >> 1
    bits = jax.lax.bitcast_convert_type(d2, jnp.int32) + (NP // 2)
    keys = (bits & ~(NP - 1)) | iota2

    kcur = keys
    S = jnp.zeros((M, TILE_N), jnp.float32)
    tot = jnp.zeros((1, TILE_N), jnp.float32)
    for kk in range(3):
        fcur = jax.lax.bitcast_convert_type(kcur, jnp.float32)
        mkf = jnp.min(fcur, axis=0, keepdims=True)
        mk = jax.lax.bitcast_convert_type(mkf, jnp.int32)
        d2k = jax.lax.bitcast_convert_type(mk & ~(NP - 1), jnp.float32)
        inv = 1.0 / (jnp.sqrt(d2k) + 1e-8)
        tot = tot + inv
        hit = kcur == mk
        S = jnp.where(hit, inv, S)
        if kk < 2:
            kcur = jnp.where(hit, jnp.int32(0x7F7FFFFF), kcur)

    fp = fp_ref[0]
    interp = jax.lax.dot_general(fp, S, (((1,), (0,)), ((), ())),
                                 preferred_element_type=jnp.float32,
                                 precision=jax.lax.Precision.HIGHEST)
    interp = interp * (1.0 / tot)
    h = (jnp.dot(w1a_ref[...], interp, preferred_element_type=jnp.float32,
                 precision=jax.lax.Precision.HIGHEST)
         + jnp.dot(w1b_ref[...], feat_ref[0], preferred_element_type=jnp.float32,
                   precision=jax.lax.Precision.HIGHEST)
         + b1_ref[...])
    out_ref[0] = jnp.maximum(h, 0.0)


@jax.jit
def kernel(xyz, xyz_prev, features, features_prev, W1, b1):
    xyzt = jnp.transpose(xyz, (0, 2, 1))
    w1a = W1[:, :C_PREV]
    w1b = W1[:, C_PREV:]
    b1c = b1[:, None]

    grid = (B, N // TILE_N)
    out = pl.pallas_call(
        _fused_body,
        grid=grid,
        in_specs=[
            pl.BlockSpec((1, 3, TILE_N), lambda b, t: (b, 0, t)),
            pl.BlockSpec((1, M, 3), lambda b, t: (b, 0, 0)),
            pl.BlockSpec((1, C_FEAT, TILE_N), lambda b, t: (b, 0, t)),
            pl.BlockSpec((1, C_PREV, M), lambda b, t: (b, 0, 0)),
            pl.BlockSpec((C_OUT, C_PREV), lambda b, t: (0, 0)),
            pl.BlockSpec((C_OUT, C_FEAT), lambda b, t: (0, 0)),
            pl.BlockSpec((C_OUT, 1), lambda b, t: (0, 0)),
        ],
        out_specs=pl.BlockSpec((1, C_OUT, TILE_N), lambda b, t: (b, 0, t)),
        out_shape=jax.ShapeDtypeStruct((B, C_OUT, N), jnp.float32),
    )(xyzt, xyz_prev, features, features_prev, w1a, w1b, b1c)
    return out


# speedup vs baseline: 1.6382x; 1.6382x over previous
import functools

import jax
import jax.numpy as jnp
from jax.experimental import pallas as pl

B, N, M = 4, 16384, 1024
C_FEAT = 64
C_PREV = 64
C_OUT = 64

TILE_N = 1024
INF = 3.0e38


def _fused_body(xyzt_ref, xyzp_ref, feat_ref, fp_ref, w1a_ref, w1b_ref,
                b1_ref, out_ref):
    q = xyzt_ref[0]
    k = xyzp_ref[0]

    d2 = jnp.zeros((M, TILE_N), jnp.float32)
    for c in range(3):
        diff = k[:, c:c + 1] - q[c:c + 1, :]
        d2 = d2 + diff * diff

    NP = M // 2
    iota2 = jax.lax.broadcasted_iota(jnp.int32, (M, TILE_N), 0) >> 1
    bits = jax.lax.bitcast_convert_type(d2, jnp.int32) + (NP // 2)
    keys = (bits & ~(NP - 1)) | iota2

    kcur = keys
    S = jnp.zeros((M, TILE_N), jnp.float32)
    tot = jnp.zeros((1, TILE_N), jnp.float32)
    for kk in range(3):
        fcur = jax.lax.bitcast_convert_type(kcur, jnp.float32)
        mkf = jnp.min(fcur, axis=0, keepdims=True)
        mk = jax.lax.bitcast_convert_type(mkf, jnp.int32)
        d2k = jax.lax.bitcast_convert_type(mk & ~(NP - 1), jnp.float32)
        inv = 1.0 / (jnp.sqrt(d2k) + 1e-8)
        tot = tot + inv
        hit = kcur == mk
        S = jnp.where(hit, inv, S)
        if kk < 2:
            kcur = jnp.where(hit, jnp.int32(0x7F7FFFFF), kcur)

    fp = fp_ref[0]
    interp = jax.lax.dot_general(fp, S, (((1,), (0,)), ((), ())),
                                 preferred_element_type=jnp.float32)
    interp = interp * (1.0 / tot)
    h = (jnp.dot(w1a_ref[...], interp, preferred_element_type=jnp.float32)
         + jnp.dot(w1b_ref[...], feat_ref[0], preferred_element_type=jnp.float32)
         + b1_ref[...])
    out_ref[0] = jnp.maximum(h, 0.0)


@jax.jit
def kernel(xyz, xyz_prev, features, features_prev, W1, b1):
    xyzt = jnp.transpose(xyz, (0, 2, 1))
    w1a = W1[:, :C_PREV]
    w1b = W1[:, C_PREV:]
    b1c = b1[:, None]

    grid = (B, N // TILE_N)
    out = pl.pallas_call(
        _fused_body,
        grid=grid,
        in_specs=[
            pl.BlockSpec((1, 3, TILE_N), lambda b, t: (b, 0, t)),
            pl.BlockSpec((1, M, 3), lambda b, t: (b, 0, 0)),
            pl.BlockSpec((1, C_FEAT, TILE_N), lambda b, t: (b, 0, t)),
            pl.BlockSpec((1, C_PREV, M), lambda b, t: (b, 0, 0)),
            pl.BlockSpec((C_OUT, C_PREV), lambda b, t: (0, 0)),
            pl.BlockSpec((C_OUT, C_FEAT), lambda b, t: (0, 0)),
            pl.BlockSpec((C_OUT, 1), lambda b, t: (0, 0)),
        ],
        out_specs=pl.BlockSpec((1, C_OUT, TILE_N), lambda b, t: (b, 0, t)),
        out_shape=jax.ShapeDtypeStruct((B, C_OUT, N), jnp.float32),
    )(xyzt, xyz_prev, features, features_prev, w1a, w1b, b1c)
    return out
